# baseline (speedup 1.0000x reference)
import functools

import jax
import jax.numpy as jnp
from jax.experimental import pallas as pl
from jax.experimental.pallas import tpu as pltpu

B = 128; N = 100; DF = 64; G = 128; D = 512; H = 8; DH = 64; A = 16; NH = 52; FF = 2048
S = 112
KIN = DF + 1 + G
OUT = A + A * N + NH + 1
NBLK = 8

f32 = jnp.float32
bf16 = jnp.bfloat16


def _body(ent_ref, gc_ref, em_ref, tm_ref, pm_ref, cm_ref,
          win_ref, bin_ref, wg_ref, b1_ref, b2_ref, wt_ref, wcard_ref, wval_ref,
          wq_hbm, wk_hbm, wv_hbm, wo_hbm, w1_hbm, w2_hbm, wp_hbm,
          out_ref,
          stage, wfull_bf, wqkv_bf, wo_bf, w1_bf, w2_bf, wbig_bf,
          gall, hall, sem):
    nb = ent_ref.shape[0]
    i = pl.program_id(0)

    @pl.when(i == 0)
    def _load_weights():
        wfull_bf[0:DF, :] = win_ref[...].astype(bf16)
        wfull_bf[DF:DF + 1, :] = bin_ref[...].astype(bf16)
        wfull_bf[DF + 1:KIN, :] = wg_ref[...].astype(bf16)
        def dma(src, dst):
            cp = pltpu.make_async_copy(src, dst, sem)
            cp.start()
            cp.wait()

        dma(wq_hbm.at[:, :], stage.at[:, 0:D])
        wqkv_bf[:, 0:D] = (stage[:, 0:D] * 0.125).astype(bf16)
        dma(wk_hbm.at[:, :], stage.at[:, 0:D])
        wqkv_bf[:, D:2 * D] = stage[:, 0:D].astype(bf16)
        dma(wv_hbm.at[:, :], stage.at[:, 0:D])
        wqkv_bf[:, 2 * D:3 * D] = stage[:, 0:D].astype(bf16)
        dma(wo_hbm.at[:, :], stage.at[:, 0:D])
        wo_bf[...] = stage[:, 0:D].astype(bf16)
        dma(w1_hbm.at[:, :], stage.at[:, :])
        w1_bf[...] = stage[...].astype(bf16)
        for j in range(FF // D):
            dma(w2_hbm.at[pl.ds(j * D, D), :], stage.at[:, 0:D])
            w2_bf[pl.ds(j * D, D), :] = stage[:, 0:D].astype(bf16)
        for j in range(4):
            dma(wp_hbm.at[:, pl.ds(j * 2048, 2048)], stage.at[:, :])
            wbig_bf[:, pl.ds(j * 2048, 2048)] = stage[...].astype(bf16)
        wbig_bf[:, A * D:A * D + A] = wt_ref[...].astype(bf16)
        wbig_bf[:, A * D + A:A * D + A + 1] = wval_ref[...].astype(bf16)

    entb = jnp.concatenate(
        [ent_ref[...].astype(bf16), jnp.zeros((nb, S - N, DF), bf16)],
        axis=1)
    rows3 = jax.lax.broadcasted_iota(jnp.int32, (nb, S, 1), 1)
    ones_col = (rows3 < N).astype(bf16)
    gcol = (rows3 == N).astype(bf16) * gc_ref[...].astype(bf16)[:, None, :]
    x2 = jnp.concatenate([entb, ones_col, gcol],
                         axis=2).reshape(nb * S, KIN)
    seq = jnp.dot(x2, wfull_bf[...], preferred_element_type=f32)

    sb = seq.astype(bf16)
    qkv = jnp.dot(sb, wqkv_bf[...], preferred_element_type=f32)
    q3 = qkv[:, :D].astype(bf16).reshape(nb, S, D)
    k3 = qkv[:, D:2 * D].astype(bf16).reshape(nb, S, D)
    v3 = qkv[:, 2 * D:].astype(bf16).reshape(nb, S, D)

    em = em_ref[...]
    keyb = jnp.concatenate(
        [jnp.where(em, 0.0, -1e9),
         jnp.zeros((nb, 1), f32),
         jnp.full((nb, S - N - 1), -1e9, f32)], axis=1)

    ctxs = []
    for hh in range(H):
        qh = q3[:, :, hh * DH:(hh + 1) * DH]
        kh = k3[:, :, hh * DH:(hh + 1) * DH]
        vh = v3[:, :, hh * DH:(hh + 1) * DH]
        s = jax.lax.dot_general(qh, kh, (((2,), (2,)), ((0,), (0,))),
                                preferred_element_type=f32)
        e = jnp.exp(s + keyb[:, None, :]).astype(bf16)
        vaug = jnp.concatenate([vh, jnp.ones((nb, S, 1), bf16)], axis=2)
        ctx = jax.lax.dot_general(e, vaug,
                                  (((2,), (1,)), ((0,), (0,))),
                                  preferred_element_type=f32)
        r = 1.0 / ctx[:, :, DH:DH + 1]
        ctxs.append((ctx[:, :, :DH] * r).astype(bf16))
    o2 = jnp.concatenate(ctxs, axis=2).reshape(nb * S, D)
    seq = seq + jnp.dot(o2, wo_bf[...], preferred_element_type=f32)

    sb2 = seq.astype(bf16)
    ff1 = jnp.maximum(
        jnp.dot(sb2, w1_bf[...], preferred_element_type=f32) + b1_ref[...],
        0.0).astype(bf16)
    seq = seq + jnp.dot(ff1, w2_bf[...], preferred_element_type=f32) + b2_ref[...]

    seq3 = seq.reshape(nb, S, D)
    gall[pl.ds(i * nb, nb), :] = seq3[:, N, :].astype(bf16)
    hall[pl.ds(i * nb, nb), :, :] = seq3[:, :N, :].astype(bf16)

    @pl.when(i == (B // nb) - 1)
    def _heads():
        gb = gall[...]
        qbig = jnp.dot(gb, wbig_bf[...],
                       preferred_element_type=f32)
        tl = qbig[:, A * D:A * D + A]
        out_ref[:, 0:A] = jnp.where(tm_ref[...], tl, -1e9)
        out_ref[:, OUT - 1: OUT] = qbig[:, A * D + A:A * D + A + 1]

        qr = jnp.stack(
            [qbig[:, a * D:(a + 1) * D] for a in range(A)]
            + [jnp.broadcast_to(wcard_ref[...], (B, D))],
            axis=1).astype(bf16)
        ptr17 = jax.lax.dot_general(qr, hall[...],
                                    (((2,), (2,)), ((0,), (0,))),
                                    preferred_element_type=f32)
        ptr = jnp.where(pm_ref[...], ptr17[:, :A, :], -1e9)
        for a in range(A):
            out_ref[:, A + a * N: A + (a + 1) * N] = ptr[:, a, :]

        out_ref[:, A + A * N: A + A * N + NH] = jnp.where(
            cm_ref[...], ptr17[:, A, :NH], -1e9)


@functools.partial(jax.jit, static_argnames=())
def kernel(entities, global_context, W_in, b_in, W_g, Wq, Wk, Wv, Wo,
           W1, b1, W2, b2, W_type, W_ptr, w_card, w_value,
           entity_mask, type_mask, pointer_masks, card_mask):
    nb = NBLK
    grid = (B // nb,)

    def blk(i):
        return (i, 0)

    def blk3(i):
        return (i, 0, 0)

    def const2(i):
        return (0, 0)

    hbm = pl.BlockSpec(memory_space=pl.ANY)

    in_specs = [
        pl.BlockSpec((nb, N, DF), blk3),
        pl.BlockSpec((nb, G), blk),
        pl.BlockSpec((nb, N), blk),
        pl.BlockSpec((B, A), const2),
        pl.BlockSpec((B, A, N), lambda i: (0, 0, 0)),
        pl.BlockSpec((B, NH), const2),
        pl.BlockSpec((DF, D), const2),
        pl.BlockSpec((1, D), const2),
        pl.BlockSpec((G, D), const2),
        pl.BlockSpec((1, FF), const2),
        pl.BlockSpec((1, D), const2),
        pl.BlockSpec((D, A), const2),
        pl.BlockSpec((1, D), const2),
        pl.BlockSpec((D, 1), const2),
        hbm, hbm, hbm, hbm, hbm, hbm, hbm,
    ]
    out_specs = pl.BlockSpec((B, OUT), const2)
    out_shape = jax.ShapeDtypeStruct((B, OUT), f32)
    scratch_shapes = [
        pltpu.VMEM((D, FF), f32),
        pltpu.VMEM((KIN, D), bf16),
        pltpu.VMEM((D, 3 * D), bf16),
        pltpu.VMEM((D, D), bf16),
        pltpu.VMEM((D, FF), bf16),
        pltpu.VMEM((FF, D), bf16),
        pltpu.VMEM((D, A * D + A + 1), bf16),
        pltpu.VMEM((B, D), bf16),
        pltpu.VMEM((B, N, D), bf16),
        pltpu.SemaphoreType.DMA,
    ]

    return pl.pallas_call(
        _body, grid=grid, in_specs=in_specs, out_specs=out_specs,
        out_shape=out_shape, scratch_shapes=scratch_shapes,
    )(entities, global_context, entity_mask, type_mask, pointer_masks,
      card_mask, W_in, b_in.reshape(1, D), W_g, b1.reshape(1, FF),
      b2.reshape(1, D), W_type, w_card.reshape(1, D), w_value,
      Wq, Wk, Wv, Wo, W1, W2, W_ptr)


# speedup vs baseline: 1.0105x; 1.0105x over previous
import functools

import jax
import jax.numpy as jnp
from jax.experimental import pallas as pl
from jax.experimental.pallas import tpu as pltpu

B = 128; N = 100; DF = 64; G = 128; D = 512; H = 8; DH = 64; A = 16; NH = 52; FF = 2048
S = 112
KIN = DF + 1 + G
OUT = A + A * N + NH + 1
NBLK = 16

f32 = jnp.float32
bf16 = jnp.bfloat16


def _body(ent_ref, gc_ref, em_ref, tm_ref, pm_ref, cm_ref,
          win_ref, bin_ref, wg_ref, b1_ref, b2_ref, wt_ref, wcard_ref, wval_ref,
          wq_hbm, wk_hbm, wv_hbm, wo_hbm, w1_hbm, w2_hbm, wp_hbm,
          out_ref,
          stage, wfull_bf, wqkv_bf, wo_bf, w1_bf, w2_bf, wbig_bf,
          gall, hall, sem):
    nb = ent_ref.shape[0]
    i = pl.program_id(0)

    @pl.when(i == 0)
    def _load_weights():
        wfull_bf[0:DF, :] = win_ref[...].astype(bf16)
        wfull_bf[DF:DF + 1, :] = bin_ref[...].astype(bf16)
        wfull_bf[DF + 1:KIN, :] = wg_ref[...].astype(bf16)
        def dma(src, dst):
            cp = pltpu.make_async_copy(src, dst, sem)
            cp.start()
            cp.wait()

        dma(wq_hbm.at[:, :], stage.at[:, 0:D])
        wqkv_bf[:, 0:D] = (stage[:, 0:D] * 0.125).astype(bf16)
        dma(wk_hbm.at[:, :], stage.at[:, 0:D])
        wqkv_bf[:, D:2 * D] = stage[:, 0:D].astype(bf16)
        dma(wv_hbm.at[:, :], stage.at[:, 0:D])
        wqkv_bf[:, 2 * D:3 * D] = stage[:, 0:D].astype(bf16)
        dma(wo_hbm.at[:, :], stage.at[:, 0:D])
        wo_bf[...] = stage[:, 0:D].astype(bf16)
        dma(w1_hbm.at[:, :], stage.at[:, :])
        w1_bf[...] = stage[...].astype(bf16)
        for j in range(FF // D):
            dma(w2_hbm.at[pl.ds(j * D, D), :], stage.at[:, 0:D])
            w2_bf[pl.ds(j * D, D), :] = stage[:, 0:D].astype(bf16)
        for j in range(4):
            dma(wp_hbm.at[:, pl.ds(j * 2048, 2048)], stage.at[:, :])
            wbig_bf[:, pl.ds(j * 2048, 2048)] = stage[...].astype(bf16)
        wbig_bf[:, A * D:A * D + A] = wt_ref[...].astype(bf16)
        wbig_bf[:, A * D + A:A * D + A + 1] = wval_ref[...].astype(bf16)

    entb = jnp.concatenate(
        [ent_ref[...].astype(bf16), jnp.zeros((nb, S - N, DF), bf16)],
        axis=1)
    rows3 = jax.lax.broadcasted_iota(jnp.int32, (nb, S, 1), 1)
    ones_col = (rows3 < N).astype(bf16)
    gcol = (rows3 == N).astype(bf16) * gc_ref[...].astype(bf16)[:, None, :]
    x2 = jnp.concatenate([entb, ones_col, gcol],
                         axis=2).reshape(nb * S, KIN)
    seq = jnp.dot(x2, wfull_bf[...], preferred_element_type=f32)

    sb = seq.astype(bf16)
    qkv = jnp.dot(sb, wqkv_bf[...], preferred_element_type=f32)
    q3 = qkv[:, :D].astype(bf16).reshape(nb, S, D)
    k3 = qkv[:, D:2 * D].astype(bf16).reshape(nb, S, D)
    v3 = qkv[:, 2 * D:].astype(bf16).reshape(nb, S, D)

    em = em_ref[...]
    keyb = jnp.concatenate(
        [jnp.where(em, 0.0, -1e9),
         jnp.zeros((nb, 1), f32),
         jnp.full((nb, S - N - 1), -1e9, f32)], axis=1)

    ctxs = []
    for hh in range(H):
        qh = q3[:, :, hh * DH:(hh + 1) * DH]
        kh = k3[:, :, hh * DH:(hh + 1) * DH]
        vh = v3[:, :, hh * DH:(hh + 1) * DH]
        s = jax.lax.dot_general(qh, kh, (((2,), (2,)), ((0,), (0,))),
                                preferred_element_type=f32)
        e = jnp.exp(s + keyb[:, None, :]).astype(bf16)
        vaug = jnp.concatenate([vh, jnp.ones((nb, S, 1), bf16)], axis=2)
        ctx = jax.lax.dot_general(e, vaug,
                                  (((2,), (1,)), ((0,), (0,))),
                                  preferred_element_type=f32)
        r = 1.0 / ctx[:, :, DH:DH + 1]
        ctxs.append((ctx[:, :, :DH] * r).astype(bf16))
    o2 = jnp.concatenate(ctxs, axis=2).reshape(nb * S, D)
    seq = seq + jnp.dot(o2, wo_bf[...], preferred_element_type=f32)

    sb2 = seq.astype(bf16)
    ff1 = jnp.maximum(
        jnp.dot(sb2, w1_bf[...], preferred_element_type=f32) + b1_ref[...],
        0.0).astype(bf16)
    seq = seq + jnp.dot(ff1, w2_bf[...], preferred_element_type=f32) + b2_ref[...]

    seq3 = seq.reshape(nb, S, D)
    gall[pl.ds(i * nb, nb), :] = seq3[:, N, :].astype(bf16)
    hall[pl.ds(i * nb, nb), :, :] = seq3[:, :N, :].astype(bf16)

    @pl.when(i == (B // nb) - 1)
    def _heads():
        gb = gall[...]
        qbig = jnp.dot(gb, wbig_bf[...],
                       preferred_element_type=f32)
        tl = qbig[:, A * D:A * D + A]
        out_ref[:, 0:A] = jnp.where(tm_ref[...], tl, -1e9)
        out_ref[:, OUT - 1: OUT] = qbig[:, A * D + A:A * D + A + 1]

        qr = jnp.stack(
            [qbig[:, a * D:(a + 1) * D] for a in range(A)]
            + [jnp.broadcast_to(wcard_ref[...], (B, D))],
            axis=1).astype(bf16)
        ptr17 = jax.lax.dot_general(qr, hall[...],
                                    (((2,), (2,)), ((0,), (0,))),
                                    preferred_element_type=f32)
        ptr = jnp.where(pm_ref[...], ptr17[:, :A, :], -1e9)
        for a in range(A):
            out_ref[:, A + a * N: A + (a + 1) * N] = ptr[:, a, :]

        out_ref[:, A + A * N: A + A * N + NH] = jnp.where(
            cm_ref[...], ptr17[:, A, :NH], -1e9)


@functools.partial(jax.jit, static_argnames=())
def kernel(entities, global_context, W_in, b_in, W_g, Wq, Wk, Wv, Wo,
           W1, b1, W2, b2, W_type, W_ptr, w_card, w_value,
           entity_mask, type_mask, pointer_masks, card_mask):
    nb = NBLK
    grid = (B // nb,)

    def blk(i):
        return (i, 0)

    def blk3(i):
        return (i, 0, 0)

    def const2(i):
        return (0, 0)

    hbm = pl.BlockSpec(memory_space=pl.ANY)

    in_specs = [
        pl.BlockSpec((nb, N, DF), blk3),
        pl.BlockSpec((nb, G), blk),
        pl.BlockSpec((nb, N), blk),
        pl.BlockSpec((B, A), const2),
        pl.BlockSpec((B, A, N), lambda i: (0, 0, 0)),
        pl.BlockSpec((B, NH), const2),
        pl.BlockSpec((DF, D), const2),
        pl.BlockSpec((1, D), const2),
        pl.BlockSpec((G, D), const2),
        pl.BlockSpec((1, FF), const2),
        pl.BlockSpec((1, D), const2),
        pl.BlockSpec((D, A), const2),
        pl.BlockSpec((1, D), const2),
        pl.BlockSpec((D, 1), const2),
        hbm, hbm, hbm, hbm, hbm, hbm, hbm,
    ]
    out_specs = pl.BlockSpec((B, OUT), const2)
    out_shape = jax.ShapeDtypeStruct((B, OUT), f32)
    scratch_shapes = [
        pltpu.VMEM((D, FF), f32),
        pltpu.VMEM((KIN, D), bf16),
        pltpu.VMEM((D, 3 * D), bf16),
        pltpu.VMEM((D, D), bf16),
        pltpu.VMEM((D, FF), bf16),
        pltpu.VMEM((FF, D), bf16),
        pltpu.VMEM((D, A * D + A + 1), bf16),
        pltpu.VMEM((B, D), bf16),
        pltpu.VMEM((B, N, D), bf16),
        pltpu.SemaphoreType.DMA,
    ]

    return pl.pallas_call(
        _body, grid=grid, in_specs=in_specs, out_specs=out_specs,
        out_shape=out_shape, scratch_shapes=scratch_shapes,
    )(entities, global_context, entity_mask, type_mask, pointer_masks,
      card_mask, W_in, b_in.reshape(1, D), W_g, b1.reshape(1, FF),
      b2.reshape(1, D), W_type, w_card.reshape(1, D), w_value,
      Wq, Wk, Wv, Wo, W1, W2, W_ptr)
